# baseline (speedup 1.0000x reference)
import functools

import jax
import jax.numpy as jnp
from jax import lax
from jax.experimental import pallas as pl
from jax.experimental.pallas import tpu as pltpu
from jax.experimental.pallas import tpu_sc as plsc

N = 100000
D = 128
M = 25000

NUM_CORES = 2
NUM_SUBCORES = 16
NW = NUM_CORES * NUM_SUBCORES
B_PER_W = 392
HALF = B_PER_W * NW
M_PAD = 2 * HALF
CHUNKS = (112, 112, 112, 56)

_MESH = plsc.VectorSubcoreMesh(core_axis_name="c", subcore_axis_name="s")


def _make_sc_gather(offset):
    @functools.partial(
        pl.kernel,
        out_type=jax.ShapeDtypeStruct((HALF, D), jnp.float32),
        mesh=_MESH,
        scratch_types=[
            pltpu.VMEM((B_PER_W,), jnp.int32),
            pltpu.VMEM((B_PER_W, D), jnp.float32),
            pltpu.SemaphoreType.DMA,
        ],
        name=f"sc_gather_{offset}",
    )
    def _sc_gather(feat_hbm, idx_hbm, out_hbm, idx_v, rows_v, sem):
        wid = lax.axis_index("s") * NUM_CORES + lax.axis_index("c")
        base = wid * B_PER_W
        pltpu.sync_copy(idx_hbm.at[pl.ds(offset + base, B_PER_W)], idx_v)
        copies = []
        pos = 0
        for c in CHUNKS:
            copies.append(
                pltpu.async_copy(
                    feat_hbm.at[idx_v.at[pl.ds(pos, c)]],
                    rows_v.at[pl.ds(pos, c)],
                    sem,
                )
            )
            pos += c
        for c in copies:
            c.wait()
        pltpu.sync_copy(rows_v, out_hbm.at[pl.ds(base, B_PER_W)])

    return _sc_gather


_sc_gather_a = _make_sc_gather(0)
_sc_gather_b = _make_sc_gather(HALF)

_TM = 3136


def _mm_body_a(g_ref, wt_ref, b_ref, o_ref):
    o_ref[...] = (
        jnp.dot(g_ref[...], wt_ref[...], preferred_element_type=jnp.float32)
        + b_ref[...]
    )


def _mm_body_b(prev_ref, g_ref, wt_ref, b_ref, o_ref):
    del prev_ref
    o_ref[...] = (
        jnp.dot(g_ref[...], wt_ref[...], preferred_element_type=jnp.float32)
        + b_ref[...]
    )


def _tc_matmul_a(g, wt, b2):
    return pl.pallas_call(
        _mm_body_a,
        grid=(HALF // _TM,),
        in_specs=[
            pl.BlockSpec((_TM, D), lambda i: (i, 0)),
            pl.BlockSpec((D, D), lambda i: (0, 0)),
            pl.BlockSpec((1, D), lambda i: (0, 0)),
        ],
        out_specs=pl.BlockSpec((_TM, D), lambda i: (i, 0)),
        out_shape=jax.ShapeDtypeStruct((M, D), jnp.float32),
    )(g, wt, b2)


def _tc_matmul_b(prev, g, wt, b2):
    nb = HALF // _TM
    return pl.pallas_call(
        _mm_body_b,
        grid=(nb,),
        in_specs=[
            pl.BlockSpec((8, D), lambda i: (0, 0)),
            pl.BlockSpec((_TM, D), lambda i: (i, 0)),
            pl.BlockSpec((D, D), lambda i: (0, 0)),
            pl.BlockSpec((1, D), lambda i: (0, 0)),
        ],
        out_specs=pl.BlockSpec((_TM, D), lambda i: (i + nb, 0)),
        out_shape=jax.ShapeDtypeStruct((M, D), jnp.float32),
        input_output_aliases={0: 0},
    )(prev, g, wt, b2)


def kernel(feat, gtensor, itensor, idi, W, b):
    del gtensor, itensor
    d_out = W.shape[0]
    d_in = W.shape[-1]
    idx_pad = jnp.concatenate([idi, jnp.zeros((M_PAD - M,), dtype=jnp.int32)])
    ga = _sc_gather_a(feat, idx_pad)
    gb = _sc_gather_b(feat, idx_pad)
    wt = W.reshape(d_out, d_in).T
    b2 = b.reshape(1, D)
    out_a = _tc_matmul_a(ga, wt, b2)
    return _tc_matmul_b(out_a, gb, wt, b2)
